# baseline (speedup 1.0000x reference)
import functools

import jax
import jax.numpy as jnp
from jax import lax
from jax.experimental import pallas as pl
from jax.experimental.pallas import tpu as pltpu
from jax.experimental.pallas import tpu_sc as plsc

BATCH = 4096
SEQ = 50
EMB = 64
VOCAB = 1000000
TOTAL = BATCH * SEQ
NC = 2
NS = 16
NW = NC * NS

WIN = 256
NWIN_FULL = VOCAB // WIN
TAIL = VOCAB - NWIN_FULL * WIN
WPW = NWIN_FULL // NW
WREM = NWIN_FULL - WPW * NW
STRIDE = WIN + 1

BPW = TOTAL // NW
CHUNK = 128
NCHUNK = BPW // CHUNK
GROUP = 5
NGROUP = NCHUNK // GROUP
GROUP_ROWS = GROUP * CHUNK


def _transpose_window(src_v, dst_v):
    iota = lax.iota(jnp.int32, 16)
    m = [(iota + d) & 15 for d in range(16)]
    riota = [iota + 16 * k for k in range(EMB // 16)]

    for k in range(EMB // 16):
        rk = [((m[d] & 1) << 6) + 16 * k + iota for d in range(16)]

        @plsc.parallel_loop(0, WIN // 16, unroll=4)
        def _sub(b, k=k, rk=rk):
            spl = jnp.full((16,), b * 16, jnp.int32)
            for d in range(16):
                gcol = spl + m[d]
                v = plsc.load_gather(src_v, [riota[k], gcol])
                q = lax.shift_right_logical(gcol, 1)
                plsc.store_scatter(dst_v, [q, rk[d]], v)


def _t_body(tin_hbm, tail_hbm, c_hbm, stg_a, stg_b, obuf_a, obuf_b,
            tbuf, sem_a, sem_b, sem_sa, sem_sb):
    wid = lax.axis_index("s") * NC + lax.axis_index("c")
    nwin = WPW + jnp.where(wid < WREM, 1, 0)

    def window_ic(g):
        return pl.multiple_of((wid + g * NW) * WIN, WIN)

    pltpu.async_copy(tin_hbm.at[:, pl.ds(window_ic(0), WIN)], stg_a, sem_a)

    def step(g, carry):
        even = lax.rem(g, 2) == 0
        ic = window_ic(g)

        @pl.when(g + 1 < nwin)
        def _prefetch():
            nic = window_ic(g + 1)
            @pl.when(even)
            def _():
                pltpu.async_copy(tin_hbm.at[:, pl.ds(nic, WIN)], stg_b, sem_b)
            @pl.when(jnp.logical_not(even))
            def _():
                pltpu.async_copy(tin_hbm.at[:, pl.ds(nic, WIN)], stg_a, sem_a)

        oc = pl.ds(pl.multiple_of(ic // 2, 8), WIN // 2)

        @pl.when(even)
        def _even():
            pltpu.make_async_copy(tin_hbm.at[:, pl.ds(ic, WIN)],
                                  stg_a, sem_a).wait()
            @pl.when(g >= 2)
            def _():
                pltpu.make_async_copy(obuf_a, c_hbm.at[oc], sem_sa).wait()
            _transpose_window(stg_a, obuf_a)
            pltpu.async_copy(obuf_a, c_hbm.at[oc], sem_sa)

        @pl.when(jnp.logical_not(even))
        def _odd():
            pltpu.make_async_copy(tin_hbm.at[:, pl.ds(ic, WIN)],
                                  stg_b, sem_b).wait()
            @pl.when(g >= 2)
            def _():
                pltpu.make_async_copy(obuf_b, c_hbm.at[oc], sem_sb).wait()
            _transpose_window(stg_b, obuf_b)
            pltpu.async_copy(obuf_b, c_hbm.at[oc], sem_sb)

        return carry

    lax.fori_loop(0, nwin, step, 0)

    pltpu.make_async_copy(obuf_a, c_hbm.at[pl.ds(0, WIN // 2)], sem_sa).wait()
    pltpu.make_async_copy(obuf_b, c_hbm.at[pl.ds(0, WIN // 2)], sem_sb).wait()

    @pl.when(wid == NW - 1)
    def _tail():
        pltpu.sync_copy(tail_hbm, tbuf)
        pltpu.sync_copy(tbuf, c_hbm.at[pl.ds(NWIN_FULL * WIN // 2, TAIL // 2)])


@jax.jit
def _transpose(table_t, tail32):
    mesh = plsc.VectorSubcoreMesh(core_axis_name="c", subcore_axis_name="s")
    fn = pl.kernel(
        _t_body,
        mesh=mesh,
        out_type=jax.ShapeDtypeStruct((VOCAB // 2, 2 * EMB), jnp.float32),
        scratch_types=[
            pltpu.VMEM((EMB, WIN), jnp.float32),
            pltpu.VMEM((EMB, WIN), jnp.float32),
            pltpu.VMEM((WIN // 2, 2 * EMB), jnp.float32),
            pltpu.VMEM((WIN // 2, 2 * EMB), jnp.float32),
            pltpu.VMEM((TAIL // 2, 2 * EMB), jnp.float32),
            pltpu.SemaphoreType.DMA,
            pltpu.SemaphoreType.DMA,
            pltpu.SemaphoreType.DMA,
            pltpu.SemaphoreType.DMA,
        ],
        compiler_params=pltpu.CompilerParams(use_tc_tiling_on_sc=True,
                                             needs_layout_passes=False),
    )
    return fn(table_t, tail32)


def _gather3_body(table_hbm, idx_hbm, out_hbm, idx_v, buf_a, buf_b,
                  sem_a, sem_b):
    wid = lax.axis_index("s") * NC + lax.axis_index("c")
    nb = BATCH // NW

    pltpu.sync_copy(idx_hbm.at[wid], idx_v)
    pltpu.async_copy(table_hbm.at[idx_v.at[0]], buf_a, sem_a)

    def step(b, carry):
        even = lax.rem(b, 2) == 0
        bg = wid * nb + b

        @pl.when(b + 1 < nb)
        def _prefetch():
            @pl.when(even)
            def _():
                pltpu.async_copy(table_hbm.at[idx_v.at[b + 1]], buf_b, sem_b)
            @pl.when(jnp.logical_not(even))
            def _():
                pltpu.async_copy(table_hbm.at[idx_v.at[b + 1]], buf_a, sem_a)

        @pl.when(even)
        def _even():
            pltpu.make_async_copy(table_hbm.at[idx_v.at[b]],
                                  buf_a, sem_a).wait()
            pltpu.sync_copy(buf_a, out_hbm.at[bg])

        @pl.when(jnp.logical_not(even))
        def _odd():
            pltpu.make_async_copy(table_hbm.at[idx_v.at[b]],
                                  buf_b, sem_b).wait()
            pltpu.sync_copy(buf_b, out_hbm.at[bg])

        return carry

    lax.fori_loop(0, nb, step, 0)


@jax.jit
def _gather3(table_lin, idx3d):
    mesh = plsc.VectorSubcoreMesh(core_axis_name="c", subcore_axis_name="s")
    fn = pl.kernel(
        _gather3_body,
        mesh=mesh,
        out_type=jax.ShapeDtypeStruct((BATCH, SEQ, EMB), jnp.float32),
        scratch_types=[
            pltpu.VMEM((BATCH // NW, SEQ), jnp.int32),
            pltpu.VMEM((SEQ, EMB), jnp.float32),
            pltpu.VMEM((SEQ, EMB), jnp.float32),
            pltpu.SemaphoreType.DMA,
            pltpu.SemaphoreType.DMA,
        ],
        compiler_params=pltpu.CompilerParams(use_tc_tiling_on_sc=False),
    )
    return fn(table_lin, idx3d)


def _gather_body(table_hbm, idx_hbm, out_hbm, idx_v, rows_v, sem):
    wid = lax.axis_index("s") * NC + lax.axis_index("c")
    base_row = wid * BPW

    pltpu.sync_copy(idx_hbm.at[wid], idx_v)

    def group(g, carry):
        copies = []
        for j in range(GROUP):
            copies.append(pltpu.async_copy(
                table_hbm.at[idx_v.at[g * GROUP + j]],
                rows_v.at[pl.ds(j * CHUNK, CHUNK)],
                sem))
        for c in copies:
            c.wait()
        pltpu.sync_copy(rows_v,
                        out_hbm.at[pl.ds(base_row + g * GROUP_ROWS, GROUP_ROWS)])
        return carry

    lax.fori_loop(0, NGROUP, group, 0)


@jax.jit
def _gather(table_lin, idx3d):
    mesh = plsc.VectorSubcoreMesh(core_axis_name="c", subcore_axis_name="s")
    fn = pl.kernel(
        _gather_body,
        mesh=mesh,
        out_type=jax.ShapeDtypeStruct((TOTAL, EMB), jnp.float32),
        scratch_types=[
            pltpu.VMEM((NCHUNK, CHUNK), jnp.int32),
            pltpu.VMEM((GROUP_ROWS, EMB), jnp.float32),
            pltpu.SemaphoreType.DMA,
        ],
        compiler_params=pltpu.CompilerParams(use_tc_tiling_on_sc=False),
    )
    return fn(table_lin, idx3d)


def kernel(image_features, captions, embedding_weight):
    table_t = jnp.transpose(embedding_weight)
    tail32 = lax.slice(embedding_weight, (NWIN_FULL * WIN, 0),
                       (VOCAB, EMB)).reshape(TAIL // 2, 2 * EMB)
    table_c = _transpose(table_t, tail32)
    table_lin = table_c.reshape(VOCAB, EMB)
    idx3d = captions.astype(jnp.int32).reshape(NW, NCHUNK, CHUNK)
    out = _gather(table_lin, idx3d)
    return out.reshape(BATCH, SEQ, EMB)


# speedup vs baseline: 1.2109x; 1.2109x over previous
import functools

import jax
import jax.numpy as jnp
from jax import lax
from jax.experimental import pallas as pl
from jax.experimental.pallas import tpu as pltpu
from jax.experimental.pallas import tpu_sc as plsc

BATCH = 4096
SEQ = 50
EMB = 64
VOCAB = 1000000
TOTAL = BATCH * SEQ
NC = 2
NS = 16
NW = NC * NS

WIN = 128
NWIN_FULL = VOCAB // WIN
TAIL = VOCAB - NWIN_FULL * WIN
WPW = NWIN_FULL // NW
WREM = NWIN_FULL - WPW * NW
STRIDE = WIN + 1

BPW = TOTAL // NW
CHUNK = 128
NCHUNK = BPW // CHUNK
GROUP = 5
NGROUP = NCHUNK // GROUP
GROUP_ROWS = GROUP * CHUNK


def _transpose_window(src_v, dst_v):
    iota = lax.iota(jnp.int32, 16)
    m = [(iota + d) & 15 for d in range(16)]
    riota = [iota + 16 * k for k in range(EMB // 16)]

    for k in range(EMB // 16):
        rk = [((m[d] & 1) << 6) + 16 * k + iota for d in range(16)]

        @plsc.parallel_loop(0, WIN // 16, unroll=4)
        def _sub(b, k=k, rk=rk):
            spl = jnp.full((16,), b * 16, jnp.int32)
            for d in range(16):
                gcol = spl + m[d]
                v = plsc.load_gather(src_v, [riota[k], gcol])
                q = lax.shift_right_logical(gcol, 1)
                plsc.store_scatter(dst_v, [q, rk[d]], v)


def _t_body(tin_hbm, tail_hbm, c_hbm, stg_a, stg_b, obuf_a, obuf_b,
            tbuf, sem_a, sem_b, sem_sa, sem_sb):
    wid = lax.axis_index("s") * NC + lax.axis_index("c")
    nwin = WPW + jnp.where(wid < WREM, 1, 0)

    def window_ic(g):
        return pl.multiple_of((wid + g * NW) * WIN, WIN)

    pltpu.async_copy(tin_hbm.at[:, pl.ds(window_ic(0), WIN)], stg_a, sem_a)

    def step(g, carry):
        even = lax.rem(g, 2) == 0
        ic = window_ic(g)

        @pl.when(g + 1 < nwin)
        def _prefetch():
            nic = window_ic(g + 1)
            @pl.when(even)
            def _():
                pltpu.async_copy(tin_hbm.at[:, pl.ds(nic, WIN)], stg_b, sem_b)
            @pl.when(jnp.logical_not(even))
            def _():
                pltpu.async_copy(tin_hbm.at[:, pl.ds(nic, WIN)], stg_a, sem_a)

        oc = pl.ds(pl.multiple_of(ic // 2, 8), WIN // 2)

        @pl.when(even)
        def _even():
            pltpu.make_async_copy(tin_hbm.at[:, pl.ds(ic, WIN)],
                                  stg_a, sem_a).wait()
            @pl.when(g >= 2)
            def _():
                pltpu.make_async_copy(obuf_a, c_hbm.at[oc], sem_sa).wait()
            _transpose_window(stg_a, obuf_a)
            pltpu.async_copy(obuf_a, c_hbm.at[oc], sem_sa)

        @pl.when(jnp.logical_not(even))
        def _odd():
            pltpu.make_async_copy(tin_hbm.at[:, pl.ds(ic, WIN)],
                                  stg_b, sem_b).wait()
            @pl.when(g >= 2)
            def _():
                pltpu.make_async_copy(obuf_b, c_hbm.at[oc], sem_sb).wait()
            _transpose_window(stg_b, obuf_b)
            pltpu.async_copy(obuf_b, c_hbm.at[oc], sem_sb)

        return carry

    lax.fori_loop(0, nwin, step, 0)

    pltpu.make_async_copy(obuf_a, c_hbm.at[pl.ds(0, WIN // 2)], sem_sa).wait()
    pltpu.make_async_copy(obuf_b, c_hbm.at[pl.ds(0, WIN // 2)], sem_sb).wait()

    @pl.when(wid == NW - 1)
    def _tail():
        pltpu.sync_copy(tail_hbm, tbuf)
        pltpu.sync_copy(tbuf, c_hbm.at[pl.ds(NWIN_FULL * WIN // 2, TAIL // 2)])


@jax.jit
def _transpose(table_t, tail32):
    mesh = plsc.VectorSubcoreMesh(core_axis_name="c", subcore_axis_name="s")
    fn = pl.kernel(
        _t_body,
        mesh=mesh,
        out_type=jax.ShapeDtypeStruct((VOCAB // 2, 2 * EMB), jnp.float32),
        scratch_types=[
            pltpu.VMEM((EMB, WIN), jnp.float32),
            pltpu.VMEM((EMB, WIN), jnp.float32),
            pltpu.VMEM((WIN // 2, 2 * EMB), jnp.float32),
            pltpu.VMEM((WIN // 2, 2 * EMB), jnp.float32),
            pltpu.VMEM((TAIL // 2, 2 * EMB), jnp.float32),
            pltpu.SemaphoreType.DMA,
            pltpu.SemaphoreType.DMA,
            pltpu.SemaphoreType.DMA,
            pltpu.SemaphoreType.DMA,
        ],
        compiler_params=pltpu.CompilerParams(use_tc_tiling_on_sc=True,
                                             needs_layout_passes=False),
    )
    return fn(table_t, tail32)


def _gather3_body(table_hbm, idx_hbm, out_hbm, idx_v, buf_a, buf_b,
                  sem_a, sem_b):
    wid = lax.axis_index("s") * NC + lax.axis_index("c")
    nb = BATCH // NW

    pltpu.sync_copy(idx_hbm.at[wid], idx_v)
    pltpu.async_copy(table_hbm.at[idx_v.at[0]], buf_a, sem_a)

    def step(b, carry):
        even = lax.rem(b, 2) == 0
        bg = wid * nb + b

        @pl.when(b + 1 < nb)
        def _prefetch():
            @pl.when(even)
            def _():
                pltpu.async_copy(table_hbm.at[idx_v.at[b + 1]], buf_b, sem_b)
            @pl.when(jnp.logical_not(even))
            def _():
                pltpu.async_copy(table_hbm.at[idx_v.at[b + 1]], buf_a, sem_a)

        @pl.when(even)
        def _even():
            pltpu.make_async_copy(table_hbm.at[idx_v.at[b]],
                                  buf_a, sem_a).wait()
            pltpu.sync_copy(buf_a, out_hbm.at[bg])

        @pl.when(jnp.logical_not(even))
        def _odd():
            pltpu.make_async_copy(table_hbm.at[idx_v.at[b]],
                                  buf_b, sem_b).wait()
            pltpu.sync_copy(buf_b, out_hbm.at[bg])

        return carry

    lax.fori_loop(0, nb, step, 0)


@jax.jit
def _gather3(table_lin, idx3d):
    mesh = plsc.VectorSubcoreMesh(core_axis_name="c", subcore_axis_name="s")
    fn = pl.kernel(
        _gather3_body,
        mesh=mesh,
        out_type=jax.ShapeDtypeStruct((BATCH, SEQ, EMB), jnp.float32),
        scratch_types=[
            pltpu.VMEM((BATCH // NW, SEQ), jnp.int32),
            pltpu.VMEM((SEQ, EMB), jnp.float32),
            pltpu.VMEM((SEQ, EMB), jnp.float32),
            pltpu.SemaphoreType.DMA,
            pltpu.SemaphoreType.DMA,
        ],
        compiler_params=pltpu.CompilerParams(use_tc_tiling_on_sc=False),
    )
    return fn(table_lin, idx3d)


def _gather_body(table_hbm, idx_hbm, out_hbm, idx_v, rows_v, sem):
    wid = lax.axis_index("s") * NC + lax.axis_index("c")
    base_row = wid * BPW

    pltpu.sync_copy(idx_hbm.at[wid], idx_v)

    def group(g, carry):
        copies = []
        for j in range(GROUP):
            copies.append(pltpu.async_copy(
                table_hbm.at[idx_v.at[g * GROUP + j]],
                rows_v.at[pl.ds(j * CHUNK, CHUNK)],
                sem))
        for c in copies:
            c.wait()
        pltpu.sync_copy(rows_v,
                        out_hbm.at[pl.ds(base_row + g * GROUP_ROWS, GROUP_ROWS)])
        return carry

    lax.fori_loop(0, NGROUP, group, 0)


@jax.jit
def _gather(table_lin, idx3d):
    mesh = plsc.VectorSubcoreMesh(core_axis_name="c", subcore_axis_name="s")
    fn = pl.kernel(
        _gather_body,
        mesh=mesh,
        out_type=jax.ShapeDtypeStruct((TOTAL, EMB), jnp.float32),
        scratch_types=[
            pltpu.VMEM((NCHUNK, CHUNK), jnp.int32),
            pltpu.VMEM((GROUP_ROWS, EMB), jnp.float32),
            pltpu.SemaphoreType.DMA,
        ],
        compiler_params=pltpu.CompilerParams(use_tc_tiling_on_sc=False),
    )
    return fn(table_lin, idx3d)


def kernel(image_features, captions, embedding_weight):
    table_t = jnp.transpose(embedding_weight)
    tail32 = lax.slice(embedding_weight, (NWIN_FULL * WIN, 0),
                       (VOCAB, EMB)).reshape(TAIL // 2, 2 * EMB)
    table_c = _transpose(table_t, tail32)
    table_lin = table_c.reshape(VOCAB, EMB)
    idx3d = captions.astype(jnp.int32).reshape(NW, NCHUNK, CHUNK)
    out = _gather(table_lin, idx3d)
    return out.reshape(BATCH, SEQ, EMB)
